# Initial kernel scaffold; baseline (speedup 1.0000x reference)
import jax
import jax.numpy as jnp
from jax.experimental import pallas as pl


def kernel(x, indexes):
    raise NotImplementedError("write your pallas kernel here")


# speedup vs baseline: 2.9568x; 2.9568x over previous
import functools

import jax
import jax.numpy as jnp
from jax.experimental import pallas as pl


def _antecedent_block(x_ref, s_ref, o_ref, *, n_ante):
    xb = x_ref[...]
    res = None
    for a in range(n_ante):
        e = jnp.dot(xb, s_ref[a], preferred_element_type=jnp.float32,
                    precision=jax.lax.Precision.HIGHEST)
        res = e if res is None else jnp.minimum(res, e)
    o_ref[...] = res


def kernel(x, indexes):
    b, n_in, n_mf = x.shape
    r, n_ante, _ = indexes.shape
    f = n_in * n_mf
    xf = x.reshape(b, f)
    idx = (indexes[..., 0] * n_mf + indexes[..., 1]).astype(jnp.int32)
    sel = jnp.swapaxes(jax.nn.one_hot(idx.T, f, dtype=jnp.float32), 1, 2)

    blk = 2048
    out = pl.pallas_call(
        functools.partial(_antecedent_block, n_ante=n_ante),
        grid=(b // blk,),
        in_specs=[
            pl.BlockSpec((blk, f), lambda i: (i, 0)),
            pl.BlockSpec((n_ante, f, r), lambda i: (0, 0, 0)),
        ],
        out_specs=pl.BlockSpec((blk, r), lambda i: (i, 0)),
        out_shape=jax.ShapeDtypeStruct((b, r), jnp.float32),
    )(xf, sel)
    return out
